# baseline (device time: 24187 ns/iter reference)
import jax
import jax.numpy as jnp
from jax import lax
from jax.experimental import pallas as pl
from jax.experimental.pallas import tpu as pltpu

N_DEV = 32
K_CHUNKS = 4


def kernel(x, w_mat):
    m_per, k = x.shape
    n = w_mat.shape[1]
    n_per = n // N_DEV
    r_w = k // K_CHUNKS

    def body(x_ref, w_hbm, out_ref, w_vmem, y_blocks, recv_blocks,
             fetch_sems, send_sem, recv_sem):
        my = lax.axis_index("i")

        fetches = []
        for r in range(K_CHUNKS):
            cp = pltpu.make_async_copy(
                w_hbm.at[pl.ds(r * r_w, r_w), :],
                w_vmem.at[pl.ds(r * r_w, r_w), :],
                fetch_sems.at[r],
            )
            cp.start()
            fetches.append(cp)

        bar = pltpu.get_barrier_semaphore()
        for d in range(1, N_DEV):
            peer = lax.rem(my + d, N_DEV)
            pl.semaphore_signal(
                bar, inc=1,
                device_id=(peer,), device_id_type=pl.DeviceIdType.MESH,
            )

        fetches[0].wait()
        yc = jnp.dot(x_ref[:, 0:r_w], w_vmem[0:r_w, :],
                     preferred_element_type=jnp.float32)
        for r in range(1, K_CHUNKS):
            fetches[r].wait()
            yc = yc + jnp.dot(
                x_ref[:, r * r_w:(r + 1) * r_w],
                w_vmem[r * r_w:(r + 1) * r_w, :],
                preferred_element_type=jnp.float32,
            )

        c1 = 0.7978845608028654
        yc = 0.5 * yc * (1.0 + jnp.tanh(c1 * (yc + 0.044715 * yc * yc * yc)))
        y16 = yc.astype(jnp.bfloat16)
        for j in range(N_DEV):
            y_blocks[j, :, :] = y16[:, j * n_per:(j + 1) * n_per]

        recv_blocks[my, :, :] = y_blocks[my, :, :]

        pl.semaphore_wait(bar, N_DEV - 1)

        for d in range(1, N_DEV):
            peer = lax.rem(my + d, N_DEV)
            pltpu.make_async_remote_copy(
                src_ref=y_blocks.at[peer],
                dst_ref=recv_blocks.at[my],
                send_sem=send_sem,
                recv_sem=recv_sem,
                device_id=(peer,),
                device_id_type=pl.DeviceIdType.MESH,
            ).start()

        join = pltpu.make_async_remote_copy(
            src_ref=y_blocks.at[pl.ds(0, N_DEV - 1)],
            dst_ref=recv_blocks.at[pl.ds(0, N_DEV - 1)],
            send_sem=send_sem,
            recv_sem=recv_sem,
            device_id=(my,),
            device_id_type=pl.DeviceIdType.MESH,
        )
        join.wait_recv()

        out_ref[:, :] = recv_blocks[:, :, :].reshape(N_DEV * m_per, n_per).astype(
            jnp.float32
        )

        join.wait_send()

    return pl.pallas_call(
        body,
        out_shape=jax.ShapeDtypeStruct((N_DEV * m_per, n_per), jnp.float32),
        in_specs=[
            pl.BlockSpec(memory_space=pltpu.VMEM),
            pl.BlockSpec(memory_space=pltpu.MemorySpace.HBM),
        ],
        out_specs=pl.BlockSpec(memory_space=pltpu.VMEM),
        scratch_shapes=[
            pltpu.VMEM((k, n), jnp.float32),
            pltpu.VMEM((N_DEV, m_per, n_per), jnp.bfloat16),
            pltpu.VMEM((N_DEV, m_per, n_per), jnp.bfloat16),
            pltpu.SemaphoreType.DMA((K_CHUNKS,)),
            pltpu.SemaphoreType.DMA,
            pltpu.SemaphoreType.DMA,
        ],
        compiler_params=pltpu.CompilerParams(collective_id=0),
    )(x, w_mat)


# device time: 21180 ns/iter; 1.1420x vs baseline; 1.1420x over previous
import jax
import jax.numpy as jnp
from jax import lax
from jax.experimental import pallas as pl
from jax.experimental.pallas import tpu as pltpu

N_DEV = 32
K_CHUNKS = 4


def kernel(x, w_mat):
    m_per, k = x.shape
    n = w_mat.shape[1]
    n_per = n // N_DEV
    r_w = k // K_CHUNKS
    half = n // 2
    c1 = 0.7978845608028654

    def body(x_ref, w_hbm, out_ref, w_vmem, y_blocks, recv_blocks,
             fetch_sems, send_sem, recv_sem):
        my = lax.axis_index("i")

        fetches = []
        for r in range(K_CHUNKS):
            cp = pltpu.make_async_copy(
                w_hbm.at[pl.ds(r * r_w, r_w), :],
                w_vmem.at[pl.ds(r * r_w, r_w), :],
                fetch_sems.at[r],
            )
            cp.start()
            fetches.append(cp)

        bar = pltpu.get_barrier_semaphore()
        for j in range(N_DEV):
            def _sig(j=j):
                pl.semaphore_signal(
                    bar, inc=1,
                    device_id=(j,), device_id_type=pl.DeviceIdType.MESH,
                )
            pl.when(j != my)(_sig)

        def gelu(v):
            return 0.5 * v * (1.0 + jnp.tanh(c1 * (v + 0.044715 * v * v * v)))

        def scatter_and_send(y16, base):
            for b in range(half // n_per):
                j = base + b
                y_blocks[j, :, :] = y16[:, b * n_per:(b + 1) * n_per]

                def _send(j=j):
                    pltpu.make_async_remote_copy(
                        src_ref=y_blocks.at[j],
                        dst_ref=recv_blocks.at[my],
                        send_sem=send_sem,
                        recv_sem=recv_sem,
                        device_id=(j,),
                        device_id_type=pl.DeviceIdType.MESH,
                    ).start()

                def _own(j=j):
                    recv_blocks[j, :, :] = y_blocks[j, :, :]

                pl.when(j != my)(_send)
                pl.when(j == my)(_own)

        fetches[0].wait()
        ya = jnp.dot(x_ref[:, 0:r_w], w_vmem[0:r_w, 0:half],
                     preferred_element_type=jnp.float32)
        for r in range(1, K_CHUNKS):
            fetches[r].wait()
            ya = ya + jnp.dot(
                x_ref[:, r * r_w:(r + 1) * r_w],
                w_vmem[r * r_w:(r + 1) * r_w, 0:half],
                preferred_element_type=jnp.float32,
            )
        ya16 = gelu(ya).astype(jnp.bfloat16)

        pl.semaphore_wait(bar, N_DEV - 1)
        scatter_and_send(ya16, 0)

        yb = jnp.dot(x_ref[:, :], w_vmem[:, half:n],
                     preferred_element_type=jnp.float32)
        yb16 = gelu(yb).astype(jnp.bfloat16)
        scatter_and_send(yb16, N_DEV // 2)

        join = pltpu.make_async_remote_copy(
            src_ref=y_blocks.at[pl.ds(0, N_DEV - 1)],
            dst_ref=recv_blocks.at[pl.ds(0, N_DEV - 1)],
            send_sem=send_sem,
            recv_sem=recv_sem,
            device_id=(0,),
            device_id_type=pl.DeviceIdType.MESH,
        )
        join.wait_recv()

        out_ref[:, :] = recv_blocks[:, :, :].reshape(N_DEV * m_per, n_per).astype(
            jnp.float32
        )

        join.wait_send()

    return pl.pallas_call(
        body,
        out_shape=jax.ShapeDtypeStruct((N_DEV * m_per, n_per), jnp.float32),
        in_specs=[
            pl.BlockSpec(memory_space=pltpu.VMEM),
            pl.BlockSpec(memory_space=pltpu.MemorySpace.HBM),
        ],
        out_specs=pl.BlockSpec(memory_space=pltpu.VMEM),
        scratch_shapes=[
            pltpu.VMEM((k, n), jnp.float32),
            pltpu.VMEM((N_DEV, m_per, n_per), jnp.bfloat16),
            pltpu.VMEM((N_DEV, m_per, n_per), jnp.bfloat16),
            pltpu.SemaphoreType.DMA((K_CHUNKS,)),
            pltpu.SemaphoreType.DMA,
            pltpu.SemaphoreType.DMA,
        ],
        compiler_params=pltpu.CompilerParams(collective_id=0),
    )(x, w_mat)
